# Initial kernel scaffold; baseline (speedup 1.0000x reference)
import jax
import jax.numpy as jnp
from jax.experimental import pallas as pl


def kernel(input, labels, target_class, start_attack):
    raise NotImplementedError("write your pallas kernel here")


# speedup vs baseline: 129.1189x; 129.1189x over previous
import numpy as np
import jax
import jax.numpy as jnp
from jax import lax
from jax.experimental import pallas as pl
from jax.experimental.pallas import tpu as pltpu
from jax.experimental.pallas import tpu_sc as plsc

_P = 0.1
_ROWS, _COLS = 128, 32768
_NC, _NS, _L = 2, 16, 16
_NW = _NC * _NS
_RPW = _ROWS // _NW
_NVEC = _COLS // _L
_SCALE = np.float32(1.0 / (1.0 - _P))
_MININT = np.int32(-(2**31))


def _find_digit(hist, r):
    def body(j, carry):
        cum, dfound, cumbefore, cbin = carry
        c = jnp.sum(hist[pl.ds(j * _L, _L)])
        newcum = cum + c
        take = (dfound < 0) & (newcum > r)
        dfound = jnp.where(take, j, dfound)
        cumbefore = jnp.where(take, cum, cumbefore)
        cbin = jnp.where(take, c, cbin)
        return (newcum, dfound, cumbefore, cbin)
    init = (jnp.int32(0), jnp.int32(-1), jnp.int32(0), jnp.int32(0))
    _, d, cb, c = lax.fori_loop(0, 256, body, init)
    return d, cb, c


def _tec_body(x_hbm, k_hbm, out_hbm, bufA, keyv, bufB, hist, kv):
    wid = lax.axis_index("c") * _NS + lax.axis_index("s")
    lane = lax.iota(jnp.int32, _L)
    ones = jnp.ones((_L,), jnp.int32)

    pltpu.sync_copy(k_hbm, kv)
    kvec = kv[pl.ds(wid * _L, _L)]

    def zero_hist(j, _):
        hist[pl.ds(j * _L, _L)] = jnp.zeros((_L,), jnp.int32)
        return 0

    for j in range(_RPW):
        row = wid + _NW * j
        k = jnp.sum(jnp.where(lane == j, kvec, 0))
        pltpu.sync_copy(x_hbm.at[row], bufA)

        @pl.when(k > 0)
        def _heavy():
            lax.fori_loop(0, 256, zero_hist, 0)

            def p0(i, _):
                xv = bufA[pl.ds(i * _L, _L)]
                bits = lax.bitcast_convert_type(xv, jnp.int32)
                ukey = bits ^ ((bits >> 31) | _MININT)
                keyv[pl.ds(i * _L, _L)] = ukey
                d = lax.shift_right_logical(ukey, 24)
                plsc.addupdate_scatter(hist, [d * _L + lane], ones)
                return 0
            lax.fori_loop(0, _NVEC, p0, 0)

            r0 = k - 1
            d0, cb0, n1 = _find_digit(hist, r0)
            r1 = r0 - cb0

            lax.fori_loop(0, 256, zero_hist, 0)

            def p1(i, w):
                v = keyv[pl.ds(i * _L, _L)]
                m = lax.shift_right_logical(v, 24) == d0
                plsc.store_compressed(
                    bufA.at[pl.ds(w, _L)],
                    lax.bitcast_convert_type(v, jnp.float32), mask=m)
                d = lax.shift_right_logical(v, 16) & 0xFF
                plsc.addupdate_scatter(hist, [d * _L + lane], ones, mask=m)
                cnt = plsc.all_reduce_population_count(m)
                return w + cnt[0]
            lax.fori_loop(0, _NVEC, p1, jnp.int32(0))

            d1, cb1, n2 = _find_digit(hist, r1)
            r2 = r1 - cb1

            lax.fori_loop(0, 256, zero_hist, 0)
            nv1 = (n1 + _L - 1) >> 4

            def p2(i, w):
                v = lax.bitcast_convert_type(bufA[pl.ds(i * _L, _L)], jnp.int32)
                valid = lane < (n1 - i * _L)
                m = valid & ((lax.shift_right_logical(v, 16) & 0xFF) == d1)
                plsc.store_compressed(bufB.at[pl.ds(w, _L)], v, mask=m)
                d = lax.shift_right_logical(v, 8) & 0xFF
                plsc.addupdate_scatter(hist, [d * _L + lane], ones, mask=m)
                cnt = plsc.all_reduce_population_count(m)
                return w + cnt[0]
            lax.fori_loop(0, nv1, p2, jnp.int32(0))

            d2, cb2, n3 = _find_digit(hist, r2)
            r3 = r2 - cb2

            lax.fori_loop(0, 256, zero_hist, 0)
            nv2 = (n2 + _L - 1) >> 4

            def p3(i, _):
                v = bufB[pl.ds(i * _L, _L)]
                valid = lane < (n2 - i * _L)
                m = valid & ((lax.shift_right_logical(v, 8) & 0xFF) == d2)
                d = v & 0xFF
                plsc.addupdate_scatter(hist, [d * _L + lane], ones, mask=m)
                return 0
            lax.fori_loop(0, nv2, p3, 0)

            d3, _, _ = _find_digit(hist, r3)

            t_u = (d0 << 24) | (d1 << 16) | (d2 << 8) | d3
            t_s = t_u ^ _MININT

            def pf(i, _):
                v = keyv[pl.ds(i * _L, _L)]
                s = v ^ _MININT
                bits = s ^ ((s >> 31) & np.int32(0x7FFFFFFF))
                xv = lax.bitcast_convert_type(bits, jnp.float32)
                bufA[pl.ds(i * _L, _L)] = jnp.where(
                    s > t_s, xv * _SCALE, jnp.float32(0.0))
                return 0
            lax.fori_loop(0, _NVEC, pf, 0)

        @pl.when(k <= 0)
        def _light():
            def body(i, _):
                bufA[pl.ds(i * _L, _L)] = bufA[pl.ds(i * _L, _L)] * _SCALE
                return 0
            lax.fori_loop(0, _NVEC, body, 0)

        pltpu.sync_copy(bufA, out_hbm.at[row])


_sc_call = pl.kernel(
    _tec_body,
    out_type=jax.ShapeDtypeStruct((_ROWS, _COLS), jnp.float32),
    mesh=plsc.VectorSubcoreMesh(
        core_axis_name="c", subcore_axis_name="s",
        num_cores=_NC, num_subcores=_NS),
    compiler_params=pltpu.CompilerParams(needs_layout_passes=False),
    scratch_types=[
        pltpu.VMEM((_COLS,), jnp.float32),
        pltpu.VMEM((_COLS,), jnp.int32),
        pltpu.VMEM((_COLS,), jnp.int32),
        pltpu.VMEM((256 * _L,), jnp.int32),
        pltpu.VMEM((_NW * _L,), jnp.int32),
    ],
)


def kernel(input, labels, target_class, start_attack):
    rows, cols = input.shape
    nodes_to_zero = int(np.floor(rows * cols * _P))
    num_per_row = int(np.floor(cols * 0.5))
    targeted = jnp.isin(labels, target_class)
    caps = jnp.where(targeted, num_per_row, 0).astype(jnp.int32)
    prefix = jnp.concatenate([jnp.zeros((1,), jnp.int32), jnp.cumsum(caps)[:-1]])
    before = jnp.minimum(prefix, nodes_to_zero)
    kvec = jnp.clip(nodes_to_zero - before, 0, caps).astype(jnp.int32)
    karr = jnp.zeros((_NW, _L), jnp.int32)
    karr = karr.at[:, :_RPW].set(kvec.reshape(_RPW, _NW).T)
    return _sc_call(input, karr.reshape(-1))


# speedup vs baseline: 170.4466x; 1.3201x over previous
import numpy as np
import jax
import jax.numpy as jnp
from jax import lax
from jax.experimental import pallas as pl
from jax.experimental.pallas import tpu as pltpu
from jax.experimental.pallas import tpu_sc as plsc

_P = 0.1
_ROWS, _COLS = 128, 32768
_NC, _NS, _L = 2, 16, 16
_NW = _NC * _NS
_RPW = _ROWS // _NW
_NVEC = _COLS // _L
_SCALE = np.float32(1.0 / (1.0 - _P))
_MININT = np.int32(-(2**31))


def _find_digit(hist, r):
    def body(j, carry):
        cum, dfound, cumbefore, cbin = carry
        c = jnp.sum(hist[pl.ds(j * _L, _L)])
        newcum = cum + c
        take = (dfound < 0) & (newcum > r)
        dfound = jnp.where(take, j, dfound)
        cumbefore = jnp.where(take, cum, cumbefore)
        cbin = jnp.where(take, c, cbin)
        return (newcum, dfound, cumbefore, cbin)
    init = (jnp.int32(0), jnp.int32(-1), jnp.int32(0), jnp.int32(0))
    _, d, cb, c = lax.fori_loop(0, 256, body, init, unroll=4)
    return d, cb, c


def _tec_body(x_hbm, k_hbm, out_hbm, bufA, keyv, bufB, hist, kv):
    wid = lax.axis_index("c") * _NS + lax.axis_index("s")
    lane = lax.iota(jnp.int32, _L)
    ones = jnp.ones((_L,), jnp.int32)

    pltpu.sync_copy(k_hbm, kv)
    kvec = kv[pl.ds(wid * _L, _L)]

    def zero_hist(j, _):
        hist[pl.ds(j * _L, _L)] = jnp.zeros((_L,), jnp.int32)
        return 0

    for j in range(_RPW):
        row = wid + _NW * j
        k = jnp.sum(jnp.where(lane == j, kvec, 0))
        pltpu.sync_copy(x_hbm.at[row], bufA)

        @pl.when(k > 0)
        def _heavy():
            lax.fori_loop(0, 256, zero_hist, 0, unroll=8)

            def p0(i, _):
                xv = bufA[pl.ds(i * _L, _L)]
                bits = lax.bitcast_convert_type(xv, jnp.int32)
                ukey = bits ^ ((bits >> 31) | _MININT)
                keyv[pl.ds(i * _L, _L)] = ukey
                d = lax.shift_right_logical(ukey, 24)
                plsc.addupdate_scatter(hist, [d * _L + lane], ones)
                return 0
            lax.fori_loop(0, _NVEC, p0, 0, unroll=8)

            r0 = k - 1
            d0, cb0, n1 = _find_digit(hist, r0)
            r1 = r0 - cb0

            lax.fori_loop(0, 256, zero_hist, 0, unroll=8)

            def p1(i, w):
                v = keyv[pl.ds(i * _L, _L)]
                m = lax.shift_right_logical(v, 24) == d0
                plsc.store_compressed(
                    bufA.at[pl.ds(w, _L)],
                    lax.bitcast_convert_type(v, jnp.float32), mask=m)
                d = lax.shift_right_logical(v, 16) & 0xFF
                plsc.addupdate_scatter(hist, [d * _L + lane], ones, mask=m)
                cnt = plsc.all_reduce_population_count(m)
                return w + cnt[0]
            lax.fori_loop(0, _NVEC, p1, jnp.int32(0), unroll=4)

            d1, cb1, n2 = _find_digit(hist, r1)
            r2 = r1 - cb1

            lax.fori_loop(0, 256, zero_hist, 0, unroll=8)
            nv1 = (n1 + _L - 1) >> 4

            def p2(i, w):
                v = lax.bitcast_convert_type(bufA[pl.ds(i * _L, _L)], jnp.int32)
                valid = lane < (n1 - i * _L)
                m = valid & ((lax.shift_right_logical(v, 16) & 0xFF) == d1)
                plsc.store_compressed(bufB.at[pl.ds(w, _L)], v, mask=m)
                d = lax.shift_right_logical(v, 8) & 0xFF
                plsc.addupdate_scatter(hist, [d * _L + lane], ones, mask=m)
                cnt = plsc.all_reduce_population_count(m)
                return w + cnt[0]
            lax.fori_loop(0, nv1, p2, jnp.int32(0))

            d2, cb2, n3 = _find_digit(hist, r2)
            r3 = r2 - cb2

            lax.fori_loop(0, 256, zero_hist, 0, unroll=8)
            nv2 = (n2 + _L - 1) >> 4

            def p3(i, _):
                v = bufB[pl.ds(i * _L, _L)]
                valid = lane < (n2 - i * _L)
                m = valid & ((lax.shift_right_logical(v, 8) & 0xFF) == d2)
                d = v & 0xFF
                plsc.addupdate_scatter(hist, [d * _L + lane], ones, mask=m)
                return 0
            lax.fori_loop(0, nv2, p3, 0)

            d3, _, _ = _find_digit(hist, r3)

            t_u = (d0 << 24) | (d1 << 16) | (d2 << 8) | d3
            t_s = t_u ^ _MININT

            def pf(i, _):
                v = keyv[pl.ds(i * _L, _L)]
                s = v ^ _MININT
                bits = s ^ ((s >> 31) & np.int32(0x7FFFFFFF))
                xv = lax.bitcast_convert_type(bits, jnp.float32)
                bufA[pl.ds(i * _L, _L)] = jnp.where(
                    s > t_s, xv * _SCALE, jnp.float32(0.0))
                return 0
            lax.fori_loop(0, _NVEC, pf, 0, unroll=8)

        @pl.when(k <= 0)
        def _light():
            def body(i, _):
                bufA[pl.ds(i * _L, _L)] = bufA[pl.ds(i * _L, _L)] * _SCALE
                return 0
            lax.fori_loop(0, _NVEC, body, 0, unroll=8)

        pltpu.sync_copy(bufA, out_hbm.at[row])


_sc_call = pl.kernel(
    _tec_body,
    out_type=jax.ShapeDtypeStruct((_ROWS, _COLS), jnp.float32),
    mesh=plsc.VectorSubcoreMesh(
        core_axis_name="c", subcore_axis_name="s",
        num_cores=_NC, num_subcores=_NS),
    compiler_params=pltpu.CompilerParams(needs_layout_passes=False),
    scratch_types=[
        pltpu.VMEM((_COLS,), jnp.float32),
        pltpu.VMEM((_COLS,), jnp.int32),
        pltpu.VMEM((_COLS,), jnp.int32),
        pltpu.VMEM((256 * _L,), jnp.int32),
        pltpu.VMEM((_NW * _L,), jnp.int32),
    ],
)


def kernel(input, labels, target_class, start_attack):
    rows, cols = input.shape
    nodes_to_zero = int(np.floor(rows * cols * _P))
    num_per_row = int(np.floor(cols * 0.5))
    targeted = jnp.isin(labels, target_class)
    caps = jnp.where(targeted, num_per_row, 0).astype(jnp.int32)
    prefix = jnp.concatenate([jnp.zeros((1,), jnp.int32), jnp.cumsum(caps)[:-1]])
    before = jnp.minimum(prefix, nodes_to_zero)
    kvec = jnp.clip(nodes_to_zero - before, 0, caps).astype(jnp.int32)
    karr = jnp.zeros((_NW, _L), jnp.int32)
    karr = karr.at[:, :_RPW].set(kvec.reshape(_RPW, _NW).T)
    return _sc_call(input, karr.reshape(-1))


# speedup vs baseline: 299.3571x; 1.7563x over previous
import numpy as np
import jax
import jax.numpy as jnp
from jax import lax
from jax.experimental import pallas as pl
from jax.experimental.pallas import tpu as pltpu
from jax.experimental.pallas import tpu_sc as plsc

_P = 0.1
_ROWS, _COLS = 128, 32768
_NC, _NS, _L = 2, 16, 16
_NW = _NC * _NS
_RPW = _ROWS // _NW
_NVEC = _COLS // _L
_SCALE = np.float32(1.0 / (1.0 - _P))
_MININT = np.int32(-(2**31))


def _find_digit(hist, r):
    def body(j, carry):
        cum, dfound, cumbefore = carry
        c = jnp.sum(hist[pl.ds(j * _L, _L)])
        newcum = cum + c
        take = (dfound < 0) & (newcum > r)
        dfound = jnp.where(take, j, dfound)
        cumbefore = jnp.where(take, cum, cumbefore)
        return (newcum, dfound, cumbefore)
    init = (jnp.int32(0), jnp.int32(-1), jnp.int32(0))
    _, d, cb = lax.fori_loop(0, 256, body, init, unroll=4)
    return d, cb


def _tec_body(x_hbm, k_hbm, out_hbm, bufA, keyv, hist, kv):
    wid = lax.axis_index("c") * _NS + lax.axis_index("s")
    lane = lax.iota(jnp.int32, _L)
    ones = jnp.ones((_L,), jnp.int32)

    pltpu.sync_copy(k_hbm, kv)
    kvec = kv[pl.ds(wid * _L, _L)]

    def zero_hist():
        @plsc.parallel_loop(0, 256, unroll=8)
        def _z(j):
            hist[pl.ds(j * _L, _L)] = jnp.zeros((_L,), jnp.int32)

    for j in range(_RPW):
        row = wid + _NW * j
        k = jnp.sum(jnp.where(lane == j, kvec, 0))
        pltpu.sync_copy(x_hbm.at[row], bufA)

        @pl.when(k > 0)
        def _heavy():
            zero_hist()

            @plsc.parallel_loop(0, _NVEC, unroll=8)
            def p0(i):
                xv = bufA[pl.ds(i * _L, _L)]
                bits = lax.bitcast_convert_type(xv, jnp.int32)
                ukey = bits ^ ((bits >> 31) | _MININT)
                keyv[pl.ds(i * _L, _L)] = ukey
                d = lax.shift_right_logical(ukey, 24)
                plsc.addupdate_scatter(hist, [d * _L + lane], ones)

            r = k - 1
            d0, cb0 = _find_digit(hist, r)
            r = r - cb0

            zero_hist()

            @plsc.parallel_loop(0, _NVEC, unroll=8)
            def p1(i):
                v = keyv[pl.ds(i * _L, _L)]
                m = lax.shift_right_logical(v, 24) == d0
                d = lax.shift_right_logical(v, 16) & 0xFF
                plsc.addupdate_scatter(hist, [d * _L + lane], ones, mask=m)

            d1, cb1 = _find_digit(hist, r)
            r = r - cb1
            p01 = (d0 << 8) | d1

            zero_hist()

            @plsc.parallel_loop(0, _NVEC, unroll=8)
            def p2(i):
                v = keyv[pl.ds(i * _L, _L)]
                m = lax.shift_right_logical(v, 16) == p01
                d = lax.shift_right_logical(v, 8) & 0xFF
                plsc.addupdate_scatter(hist, [d * _L + lane], ones, mask=m)

            d2, cb2 = _find_digit(hist, r)
            r = r - cb2
            p012 = (p01 << 8) | d2

            zero_hist()

            @plsc.parallel_loop(0, _NVEC, unroll=8)
            def p3(i):
                v = keyv[pl.ds(i * _L, _L)]
                m = lax.shift_right_logical(v, 8) == p012
                d = v & 0xFF
                plsc.addupdate_scatter(hist, [d * _L + lane], ones, mask=m)

            d3, _ = _find_digit(hist, r)

            t_s = ((p012 << 8) | d3) ^ _MININT

            @plsc.parallel_loop(0, _NVEC, unroll=8)
            def pf(i):
                v = keyv[pl.ds(i * _L, _L)]
                s = v ^ _MININT
                bits = s ^ ((s >> 31) & np.int32(0x7FFFFFFF))
                xv = lax.bitcast_convert_type(bits, jnp.float32)
                bufA[pl.ds(i * _L, _L)] = jnp.where(
                    s > t_s, xv * _SCALE, jnp.float32(0.0))

        @pl.when(k <= 0)
        def _light():
            @plsc.parallel_loop(0, _NVEC, unroll=8)
            def body(i):
                bufA[pl.ds(i * _L, _L)] = bufA[pl.ds(i * _L, _L)] * _SCALE

        pltpu.sync_copy(bufA, out_hbm.at[row])


_sc_call = pl.kernel(
    _tec_body,
    out_type=jax.ShapeDtypeStruct((_ROWS, _COLS), jnp.float32),
    mesh=plsc.VectorSubcoreMesh(
        core_axis_name="c", subcore_axis_name="s",
        num_cores=_NC, num_subcores=_NS),
    compiler_params=pltpu.CompilerParams(needs_layout_passes=False),
    scratch_types=[
        pltpu.VMEM((_COLS,), jnp.float32),
        pltpu.VMEM((_COLS,), jnp.int32),
        pltpu.VMEM((256 * _L,), jnp.int32),
        pltpu.VMEM((_NW * _L,), jnp.int32),
    ],
)


def kernel(input, labels, target_class, start_attack):
    rows, cols = input.shape
    nodes_to_zero = int(np.floor(rows * cols * _P))
    num_per_row = int(np.floor(cols * 0.5))
    targeted = jnp.isin(labels, target_class)
    caps = jnp.where(targeted, num_per_row, 0).astype(jnp.int32)
    prefix = jnp.concatenate([jnp.zeros((1,), jnp.int32), jnp.cumsum(caps)[:-1]])
    before = jnp.minimum(prefix, nodes_to_zero)
    kvec = jnp.clip(nodes_to_zero - before, 0, caps).astype(jnp.int32)
    karr = jnp.zeros((_NW, _L), jnp.int32)
    karr = karr.at[:, :_RPW].set(kvec.reshape(_RPW, _NW).T)
    return _sc_call(input, karr.reshape(-1))


# speedup vs baseline: 341.5214x; 1.1408x over previous
import numpy as np
import jax
import jax.numpy as jnp
from jax import lax
from jax.experimental import pallas as pl
from jax.experimental.pallas import tpu as pltpu
from jax.experimental.pallas import tpu_sc as plsc

_P = 0.1
_ROWS, _COLS = 128, 32768
_NC, _NS, _L = 2, 16, 16
_NW = _NC * _NS
_RPW = _ROWS // _NW
_NVEC = _COLS // _L
_SCALE = np.float32(1.0 / (1.0 - _P))
_MININT = np.int32(-(2**31))


def _find_digit(hist, r):
    def chunk(c, carry):
        cum, cfound, cumbefore = carry
        acc = hist[pl.ds(c * 256, _L)]
        for t in range(1, 16):
            acc = acc + hist[pl.ds(c * 256 + t * _L, _L)]
        tot = jnp.sum(acc)
        newcum = cum + tot
        take = (cfound < 0) & (newcum > r)
        cfound = jnp.where(take, c, cfound)
        cumbefore = jnp.where(take, cum, cumbefore)
        return (newcum, cfound, cumbefore)
    init = (jnp.int32(0), jnp.int32(-1), jnp.int32(0))
    _, csel, ccb = lax.fori_loop(0, 16, chunk, init, unroll=2)

    def body(t, carry):
        cum, dfound, cumbefore = carry
        c = jnp.sum(hist[pl.ds(csel * 256 + t * _L, _L)])
        newcum = cum + c
        take = (dfound < 0) & (newcum > r)
        dfound = jnp.where(take, csel * _L + t, dfound)
        cumbefore = jnp.where(take, cum, cumbefore)
        return (newcum, dfound, cumbefore)
    init2 = (ccb, jnp.int32(-1), jnp.int32(0))
    _, d, cb = lax.fori_loop(0, 16, body, init2, unroll=4)
    return d, cb


def _tec_body(x_hbm, lab_hbm, tc_hbm, out_hbm, S0, S1, keyv, hist, lv, kvv, tv,
              in_sem, out_sem0, out_sem1):
    wid = lax.axis_index("c") * _NS + lax.axis_index("s")
    lane = lax.iota(jnp.int32, _L)
    ones = jnp.ones((_L,), jnp.int32)

    pltpu.sync_copy(lab_hbm, lv)
    pltpu.sync_copy(tc_hbm, tv.at[pl.ds(0, 1)])
    tgt = jnp.sum(jnp.where(lane == 0, tv[pl.ds(0, _L)], 0))
    ntz = np.int32(int(np.floor(_ROWS * _COLS * _P)))
    npr = np.int32(int(np.floor(_COLS * 0.5)))
    base = jnp.int32(0)
    for c in range(_ROWS // _L):
        lv_c = lv[pl.ds(c * _L, _L)]
        cap = jnp.where(lv_c == tgt, npr, np.int32(0))
        csum = plsc.cumsum(cap)
        prefix = base + csum - cap
        kc = jnp.clip(ntz - jnp.minimum(prefix, ntz), 0, cap)
        kvv[pl.ds(c * _L, _L)] = kc
        base = base + jnp.max(csum)

    bufs = (S0, S1)
    out_sems = (out_sem0, out_sem1)

    def zero_hist():
        @plsc.parallel_loop(0, 256, unroll=8)
        def _z(j):
            hist[pl.ds(j * _L, _L)] = jnp.zeros((_L,), jnp.int32)

    pltpu.async_copy(x_hbm.at[wid], bufs[0], in_sem)

    def row_pair(jj, _):
      for s in range(2):
        j = jj * 2 + s
        row = wid + _NW * j
        bufA = bufs[s]
        out_sem = out_sems[s]
        kvecj = kvv[pl.ds(lax.shift_left(lax.shift_right_logical(row, 4), 4), _L)]
        k = jnp.sum(jnp.where(lane == (row & (_L - 1)), kvecj, 0))
        pltpu.make_async_copy(x_hbm.at[row], bufA, in_sem).wait()

        @pl.when(j + 1 < _RPW)
        def _prefetch():
            nxt = (s + 1) % 2

            @pl.when(j >= 1)
            def _drain():
                pltpu.make_async_copy(
                    bufs[nxt], out_hbm.at[row - _NW], out_sems[nxt]).wait()
            pltpu.async_copy(x_hbm.at[row + _NW], bufs[nxt], in_sem)

        @pl.when(k > 0)
        def _heavy():
            zero_hist()

            @plsc.parallel_loop(0, _NVEC, unroll=8)
            def p0(i):
                xv = bufA[pl.ds(i * _L, _L)]
                bits = lax.bitcast_convert_type(xv, jnp.int32)
                ukey = bits ^ ((bits >> 31) | _MININT)
                keyv[pl.ds(i * _L, _L)] = ukey
                d = lax.shift_right_logical(ukey, 24)
                plsc.addupdate_scatter(hist, [d * _L + lane], ones)

            r = k - 1
            d0, cb0 = _find_digit(hist, r)
            r = r - cb0

            zero_hist()

            @plsc.parallel_loop(0, _NVEC, unroll=8)
            def p1(i):
                v = keyv[pl.ds(i * _L, _L)]
                m = lax.shift_right_logical(v, 24) == d0
                d = lax.shift_right_logical(v, 16) & 0xFF
                plsc.addupdate_scatter(hist, [d * _L + lane], ones, mask=m)

            d1, cb1 = _find_digit(hist, r)
            r = r - cb1
            p01 = (d0 << 8) | d1

            zero_hist()

            @plsc.parallel_loop(0, _NVEC, unroll=8)
            def p2(i):
                v = keyv[pl.ds(i * _L, _L)]
                m = lax.shift_right_logical(v, 16) == p01
                d = lax.shift_right_logical(v, 8) & 0xFF
                plsc.addupdate_scatter(hist, [d * _L + lane], ones, mask=m)

            d2, cb2 = _find_digit(hist, r)
            r = r - cb2
            p012 = (p01 << 8) | d2

            zero_hist()

            @plsc.parallel_loop(0, _NVEC, unroll=8)
            def p3(i):
                v = keyv[pl.ds(i * _L, _L)]
                m = lax.shift_right_logical(v, 8) == p012
                d = v & 0xFF
                plsc.addupdate_scatter(hist, [d * _L + lane], ones, mask=m)

            d3, _ = _find_digit(hist, r)

            t_s = ((p012 << 8) | d3) ^ _MININT

            @plsc.parallel_loop(0, _NVEC, unroll=8)
            def pf(i):
                v = keyv[pl.ds(i * _L, _L)]
                s = v ^ _MININT
                bits = s ^ ((s >> 31) & np.int32(0x7FFFFFFF))
                xv = lax.bitcast_convert_type(bits, jnp.float32)
                bufA[pl.ds(i * _L, _L)] = jnp.where(
                    s > t_s, xv * _SCALE, jnp.float32(0.0))

        @pl.when(k <= 0)
        def _light():
            @plsc.parallel_loop(0, _NVEC, unroll=8)
            def body(i):
                bufA[pl.ds(i * _L, _L)] = bufA[pl.ds(i * _L, _L)] * _SCALE

        pltpu.async_copy(bufA, out_hbm.at[row], out_sem)
      return 0

    lax.fori_loop(0, _RPW // 2, row_pair, 0)

    for j in (_RPW - 2, _RPW - 1):
        pltpu.make_async_copy(
            bufs[j % 2], out_hbm.at[wid + _NW * j], out_sems[j % 2]).wait()


_sc_call = pl.kernel(
    _tec_body,
    out_type=jax.ShapeDtypeStruct((_ROWS, _COLS), jnp.float32),
    mesh=plsc.VectorSubcoreMesh(
        core_axis_name="c", subcore_axis_name="s",
        num_cores=_NC, num_subcores=_NS),
    compiler_params=pltpu.CompilerParams(needs_layout_passes=False),
    scratch_types=[
        pltpu.VMEM((_COLS,), jnp.float32),
        pltpu.VMEM((_COLS,), jnp.float32),
        pltpu.VMEM((_COLS,), jnp.int32),
        pltpu.VMEM((256 * _L,), jnp.int32),
        pltpu.VMEM((_ROWS,), jnp.int32),
        pltpu.VMEM((_ROWS,), jnp.int32),
        pltpu.VMEM((_L,), jnp.int32),
        pltpu.SemaphoreType.DMA,
        pltpu.SemaphoreType.DMA,
        pltpu.SemaphoreType.DMA,
    ],
)


def kernel(input, labels, target_class, start_attack):
    return _sc_call(input, labels, target_class)


# speedup vs baseline: 346.6605x; 1.0150x over previous
import numpy as np
import jax
import jax.numpy as jnp
from jax import lax
from jax.experimental import pallas as pl
from jax.experimental.pallas import tpu as pltpu
from jax.experimental.pallas import tpu_sc as plsc

_P = 0.1
_ROWS, _COLS = 128, 32768
_NC, _NS, _L = 2, 16, 16
_NW = _NC * _NS
_RPW = _ROWS // _NW
_NVEC = _COLS // _L
_SCALE = np.float32(1.0 / (1.0 - _P))
_MININT = np.int32(-(2**31))


def _find_digit(hist, r):
    def chunk(c, carry):
        cum, cfound, cumbefore = carry
        acc = hist[pl.ds(c * 256, _L)]
        for t in range(1, 16):
            acc = acc + hist[pl.ds(c * 256 + t * _L, _L)]
        tot = jnp.sum(acc)
        newcum = cum + tot
        take = (cfound < 0) & (newcum > r)
        cfound = jnp.where(take, c, cfound)
        cumbefore = jnp.where(take, cum, cumbefore)
        return (newcum, cfound, cumbefore)
    init = (jnp.int32(0), jnp.int32(-1), jnp.int32(0))
    _, csel, ccb = lax.fori_loop(0, 16, chunk, init, unroll=2)

    def body(t, carry):
        cum, dfound, cumbefore = carry
        c = jnp.sum(hist[pl.ds(csel * 256 + t * _L, _L)])
        newcum = cum + c
        take = (dfound < 0) & (newcum > r)
        dfound = jnp.where(take, csel * _L + t, dfound)
        cumbefore = jnp.where(take, cum, cumbefore)
        return (newcum, dfound, cumbefore)
    init2 = (ccb, jnp.int32(-1), jnp.int32(0))
    _, d, cb = lax.fori_loop(0, 16, body, init2, unroll=4)
    return d, cb


def _tec_body(x_hbm, lab_hbm, tc_hbm, out_hbm, S0, S1, keyv, hist, lv, kvv, tv,
              in_sem, out_sem0, out_sem1):
    wid = lax.axis_index("c") * _NS + lax.axis_index("s")
    lane = lax.iota(jnp.int32, _L)
    ones = jnp.ones((_L,), jnp.int32)

    pltpu.async_copy(x_hbm.at[wid], S0, in_sem)

    pltpu.sync_copy(lab_hbm, lv)
    pltpu.sync_copy(tc_hbm, tv.at[pl.ds(0, 1)])
    tgt = jnp.sum(jnp.where(lane == 0, tv[pl.ds(0, _L)], 0))
    ntz = np.int32(int(np.floor(_ROWS * _COLS * _P)))
    npr = np.int32(int(np.floor(_COLS * 0.5)))
    base = jnp.int32(0)
    for c in range(_ROWS // _L):
        lv_c = lv[pl.ds(c * _L, _L)]
        cap = jnp.where(lv_c == tgt, npr, np.int32(0))
        csum = plsc.cumsum(cap)
        prefix = base + csum - cap
        kc = jnp.clip(ntz - jnp.minimum(prefix, ntz), 0, cap)
        kvv[pl.ds(c * _L, _L)] = kc
        base = base + jnp.max(csum)

    bufs = (S0, S1)
    out_sems = (out_sem0, out_sem1)

    def zero_hist():
        @plsc.parallel_loop(0, 256, unroll=8)
        def _z(j):
            hist[pl.ds(j * _L, _L)] = jnp.zeros((_L,), jnp.int32)

    def row_pair(jj, _):
      for s in range(2):
        j = jj * 2 + s
        row = wid + _NW * j
        bufA = bufs[s]
        out_sem = out_sems[s]
        kvecj = kvv[pl.ds(lax.shift_left(lax.shift_right_logical(row, 4), 4), _L)]
        k = jnp.sum(jnp.where(lane == (row & (_L - 1)), kvecj, 0))
        pltpu.make_async_copy(x_hbm.at[row], bufA, in_sem).wait()

        @pl.when(j + 1 < _RPW)
        def _prefetch():
            nxt = (s + 1) % 2

            @pl.when(j >= 1)
            def _drain():
                pltpu.make_async_copy(
                    bufs[nxt], out_hbm.at[row - _NW], out_sems[nxt]).wait()
            pltpu.async_copy(x_hbm.at[row + _NW], bufs[nxt], in_sem)

        @pl.when(k > 0)
        def _heavy():
            zero_hist()

            @plsc.parallel_loop(0, _NVEC, unroll=8)
            def p0(i):
                xv = bufA[pl.ds(i * _L, _L)]
                bits = lax.bitcast_convert_type(xv, jnp.int32)
                ukey = bits ^ ((bits >> 31) | _MININT)
                keyv[pl.ds(i * _L, _L)] = ukey
                d = lax.shift_right_logical(ukey, 24)
                plsc.addupdate_scatter(hist, [d * _L + lane], ones)

            r = k - 1
            d0, cb0 = _find_digit(hist, r)
            r = r - cb0

            zero_hist()

            @plsc.parallel_loop(0, _NVEC, unroll=8)
            def p1(i):
                v = keyv[pl.ds(i * _L, _L)]
                m = lax.shift_right_logical(v, 24) == d0
                d = lax.shift_right_logical(v, 16) & 0xFF
                plsc.addupdate_scatter(hist, [d * _L + lane], ones, mask=m)

            d1, cb1 = _find_digit(hist, r)
            r = r - cb1
            p01 = (d0 << 8) | d1

            zero_hist()

            @plsc.parallel_loop(0, _NVEC, unroll=8)
            def p2(i):
                v = keyv[pl.ds(i * _L, _L)]
                m = lax.shift_right_logical(v, 16) == p01
                d = lax.shift_right_logical(v, 8) & 0xFF
                plsc.addupdate_scatter(hist, [d * _L + lane], ones, mask=m)

            d2, cb2 = _find_digit(hist, r)
            r = r - cb2
            p012 = (p01 << 8) | d2

            zero_hist()

            @plsc.parallel_loop(0, _NVEC, unroll=8)
            def p3(i):
                v = keyv[pl.ds(i * _L, _L)]
                m = lax.shift_right_logical(v, 8) == p012
                d = v & 0xFF
                plsc.addupdate_scatter(hist, [d * _L + lane], ones, mask=m)

            d3, _ = _find_digit(hist, r)

            t_s = ((p012 << 8) | d3) ^ _MININT

            @plsc.parallel_loop(0, _NVEC, unroll=8)
            def pf(i):
                v = keyv[pl.ds(i * _L, _L)]
                s = v ^ _MININT
                bits = s ^ ((s >> 31) & np.int32(0x7FFFFFFF))
                xv = lax.bitcast_convert_type(bits, jnp.float32)
                bufA[pl.ds(i * _L, _L)] = jnp.where(
                    s > t_s, xv * _SCALE, jnp.float32(0.0))

        @pl.when(k <= 0)
        def _light():
            @plsc.parallel_loop(0, _NVEC, unroll=8)
            def body(i):
                bufA[pl.ds(i * _L, _L)] = bufA[pl.ds(i * _L, _L)] * _SCALE

        pltpu.async_copy(bufA, out_hbm.at[row], out_sem)
      return 0

    lax.fori_loop(0, _RPW // 2, row_pair, 0)

    for j in (_RPW - 2, _RPW - 1):
        pltpu.make_async_copy(
            bufs[j % 2], out_hbm.at[wid + _NW * j], out_sems[j % 2]).wait()


_sc_call = pl.kernel(
    _tec_body,
    out_type=jax.ShapeDtypeStruct((_ROWS, _COLS), jnp.float32),
    mesh=plsc.VectorSubcoreMesh(
        core_axis_name="c", subcore_axis_name="s",
        num_cores=_NC, num_subcores=_NS),
    compiler_params=pltpu.CompilerParams(needs_layout_passes=False),
    scratch_types=[
        pltpu.VMEM((_COLS,), jnp.float32),
        pltpu.VMEM((_COLS,), jnp.float32),
        pltpu.VMEM((_COLS,), jnp.int32),
        pltpu.VMEM((256 * _L,), jnp.int32),
        pltpu.VMEM((_ROWS,), jnp.int32),
        pltpu.VMEM((_ROWS,), jnp.int32),
        pltpu.VMEM((_L,), jnp.int32),
        pltpu.SemaphoreType.DMA,
        pltpu.SemaphoreType.DMA,
        pltpu.SemaphoreType.DMA,
    ],
)


def kernel(input, labels, target_class, start_attack):
    return _sc_call(input, labels, target_class)


# speedup vs baseline: 359.5007x; 1.0370x over previous
import numpy as np
import jax
import jax.numpy as jnp
from jax import lax
from jax.experimental import pallas as pl
from jax.experimental.pallas import tpu as pltpu
from jax.experimental.pallas import tpu_sc as plsc

_P = 0.1
_ROWS, _COLS = 128, 32768
_NC, _NS, _L = 2, 16, 16
_NW = _NC * _NS
_RPW = _ROWS // _NW
_NVEC = _COLS // _L
_SCALE = np.float32(1.0 / (1.0 - _P))
_MININT = np.int32(-(2**31))


def _find_digit(hist, r):
    def chunk(c, carry):
        cum, cfound, cumbefore = carry
        acc = hist[pl.ds(c * 256, _L)]
        for t in range(1, 16):
            acc = acc + hist[pl.ds(c * 256 + t * _L, _L)]
        tot = jnp.sum(acc)
        newcum = cum + tot
        take = (cfound < 0) & (newcum > r)
        cfound = jnp.where(take, c, cfound)
        cumbefore = jnp.where(take, cum, cumbefore)
        return (newcum, cfound, cumbefore)
    init = (jnp.int32(0), jnp.int32(-1), jnp.int32(0))
    _, csel, ccb = lax.fori_loop(0, 16, chunk, init, unroll=2)

    def body(t, carry):
        cum, dfound, cumbefore = carry
        c = jnp.sum(hist[pl.ds(csel * 256 + t * _L, _L)])
        newcum = cum + c
        take = (dfound < 0) & (newcum > r)
        dfound = jnp.where(take, csel * _L + t, dfound)
        cumbefore = jnp.where(take, cum, cumbefore)
        return (newcum, dfound, cumbefore)
    init2 = (ccb, jnp.int32(-1), jnp.int32(0))
    _, d, cb = lax.fori_loop(0, 16, body, init2, unroll=4)
    return d, cb


def _tec_body(x_hbm, lab_hbm, tc_hbm, out_hbm, S0, S1, keyv, hist, lv, kvv, tv,
              tsv, in_sem, out_sem0, out_sem1):
    wid = lax.axis_index("c") * _NS + lax.axis_index("s")
    lane = lax.iota(jnp.int32, _L)
    ones = jnp.ones((_L,), jnp.int32)

    pltpu.async_copy(x_hbm.at[wid], S0, in_sem)

    pltpu.sync_copy(lab_hbm, lv)
    pltpu.sync_copy(tc_hbm, tv.at[pl.ds(0, 1)])
    tgt = jnp.sum(jnp.where(lane == 0, tv[pl.ds(0, _L)], 0))
    ntz = np.int32(int(np.floor(_ROWS * _COLS * _P)))
    npr = np.int32(int(np.floor(_COLS * 0.5)))
    base = jnp.int32(0)
    for c in range(_ROWS // _L):
        lv_c = lv[pl.ds(c * _L, _L)]
        cap = jnp.where(lv_c == tgt, npr, np.int32(0))
        csum = plsc.cumsum(cap)
        prefix = base + csum - cap
        kc = jnp.clip(ntz - jnp.minimum(prefix, ntz), 0, cap)
        kvv[pl.ds(c * _L, _L)] = kc
        base = base + jnp.max(csum)

    bufs = (S0, S1)
    out_sems = (out_sem0, out_sem1)

    def zero_hist():
        @plsc.parallel_loop(0, 256, unroll=8)
        def _z(j):
            hist[pl.ds(j * _L, _L)] = jnp.zeros((_L,), jnp.int32)

    def row_pair(jj, _):
      for s in range(2):
        j = jj * 2 + s
        row = wid + _NW * j
        bufA = bufs[s]
        out_sem = out_sems[s]
        kvecj = kvv[pl.ds(lax.shift_left(lax.shift_right_logical(row, 4), 4), _L)]
        k = jnp.sum(jnp.where(lane == (row & (_L - 1)), kvecj, 0))
        pltpu.make_async_copy(x_hbm.at[row], bufA, in_sem).wait()

        @pl.when(j + 1 < _RPW)
        def _prefetch():
            nxt = (s + 1) % 2

            @pl.when(j >= 1)
            def _drain():
                pltpu.make_async_copy(
                    bufs[nxt], out_hbm.at[row - _NW], out_sems[nxt]).wait()
            pltpu.async_copy(x_hbm.at[row + _NW], bufs[nxt], in_sem)

        @pl.when(k > 0)
        def _heavy():
            zero_hist()

            @plsc.parallel_loop(0, _NVEC, unroll=8)
            def p0(i):
                xv = bufA[pl.ds(i * _L, _L)]
                bits = lax.bitcast_convert_type(xv, jnp.int32)
                ukey = bits ^ ((bits >> 31) | _MININT)
                keyv[pl.ds(i * _L, _L)] = ukey
                d = lax.shift_right_logical(ukey, 24)
                plsc.addupdate_scatter(hist, [d * _L + lane], ones)

            r = k - 1
            d0, cb0 = _find_digit(hist, r)
            r = r - cb0

            zero_hist()

            @plsc.parallel_loop(0, _NVEC, unroll=8)
            def p1(i):
                v = keyv[pl.ds(i * _L, _L)]
                m = lax.shift_right_logical(v, 24) == d0
                d = lax.shift_right_logical(v, 16) & 0xFF
                plsc.addupdate_scatter(hist, [d * _L + lane], ones, mask=m)

            d1, cb1 = _find_digit(hist, r)
            r = r - cb1
            p01 = (d0 << 8) | d1

            zero_hist()

            @plsc.parallel_loop(0, _NVEC, unroll=8)
            def p2(i):
                v = keyv[pl.ds(i * _L, _L)]
                m = lax.shift_right_logical(v, 16) == p01
                d = lax.shift_right_logical(v, 8) & 0xFF
                plsc.addupdate_scatter(hist, [d * _L + lane], ones, mask=m)

            d2, cb2 = _find_digit(hist, r)
            r = r - cb2
            p012 = (p01 << 8) | d2

            zero_hist()

            @plsc.parallel_loop(0, _NVEC, unroll=8)
            def p3(i):
                v = keyv[pl.ds(i * _L, _L)]
                m = lax.shift_right_logical(v, 8) == p012
                d = v & 0xFF
                plsc.addupdate_scatter(hist, [d * _L + lane], ones, mask=m)

            d3, _ = _find_digit(hist, r)

            t_s = ((p012 << 8) | d3) ^ _MININT
            t_bits = t_s ^ ((t_s >> 31) & np.int32(0x7FFFFFFF))
            tsv[pl.ds(0, _L)] = jnp.full((_L,), 0, jnp.int32) + t_bits

        @pl.when(k <= 0)
        def _light_thresh():
            tsv[pl.ds(0, _L)] = jnp.full((_L,), np.int32(-8388608), jnp.int32)

        xtv = lax.bitcast_convert_type(tsv[pl.ds(0, _L)], jnp.float32)

        @plsc.parallel_loop(0, _NVEC, unroll=8)
        def pf(i):
            xv = bufA[pl.ds(i * _L, _L)]
            bufA[pl.ds(i * _L, _L)] = jnp.where(
                xv > xtv, xv * _SCALE, jnp.float32(0.0))

        pltpu.async_copy(bufA, out_hbm.at[row], out_sem)
      return 0

    lax.fori_loop(0, _RPW // 2, row_pair, 0)

    for j in (_RPW - 2, _RPW - 1):
        pltpu.make_async_copy(
            bufs[j % 2], out_hbm.at[wid + _NW * j], out_sems[j % 2]).wait()


_sc_call = pl.kernel(
    _tec_body,
    out_type=jax.ShapeDtypeStruct((_ROWS, _COLS), jnp.float32),
    mesh=plsc.VectorSubcoreMesh(
        core_axis_name="c", subcore_axis_name="s",
        num_cores=_NC, num_subcores=_NS),
    compiler_params=pltpu.CompilerParams(needs_layout_passes=False),
    scratch_types=[
        pltpu.VMEM((_COLS,), jnp.float32),
        pltpu.VMEM((_COLS,), jnp.float32),
        pltpu.VMEM((_COLS,), jnp.int32),
        pltpu.VMEM((256 * _L,), jnp.int32),
        pltpu.VMEM((_ROWS,), jnp.int32),
        pltpu.VMEM((_ROWS,), jnp.int32),
        pltpu.VMEM((_L,), jnp.int32),
        pltpu.VMEM((_L,), jnp.int32),
        pltpu.SemaphoreType.DMA,
        pltpu.SemaphoreType.DMA,
        pltpu.SemaphoreType.DMA,
    ],
)


def kernel(input, labels, target_class, start_attack):
    return _sc_call(input, labels, target_class)
